# baseline (speedup 1.0000x reference)
import functools

import jax
import jax.numpy as jnp
from jax import lax
from jax.experimental import pallas as pl
from jax.experimental.pallas import tpu as pltpu
from jax.experimental.pallas import tpu_sc as plsc


def _gather_rows(x_flat, tok_embed):
    n = x_flat.shape[0]
    embed = tok_embed.shape[1]
    info = plsc.get_sparse_core_info()
    nc, ns = info.num_cores, info.num_subcores
    nw = nc * ns
    rows_per_w = n // nw
    chunk = 32
    n_chunks = rows_per_w // chunk

    mesh = plsc.VectorSubcoreMesh(core_axis_name="c", subcore_axis_name="s")

    @functools.partial(
        pl.kernel,
        mesh=mesh,
        out_type=jax.ShapeDtypeStruct((n, embed), jnp.float32),
        scratch_types=[
            pltpu.VMEM((rows_per_w,), jnp.int32),
            pltpu.VMEM((chunk, embed), jnp.float32),
            pltpu.VMEM((chunk, embed), jnp.float32),
            pltpu.SemaphoreType.DMA,
            pltpu.SemaphoreType.DMA,
            pltpu.SemaphoreType.DMA,
            pltpu.SemaphoreType.DMA,
        ],
    )
    def k(x_hbm, tok_hbm, out_hbm, idx_v, rows0, rows1, g0, g1, s0, s1):
        wid = lax.axis_index("s") * nc + lax.axis_index("c")
        base = wid * rows_per_w
        pltpu.sync_copy(x_hbm.at[pl.ds(base, rows_per_w)], idx_v)
        rows = (rows0, rows1)
        gsem = (g0, g1)
        ssem = (s0, s1)
        gathers = [None] * n_chunks
        stores = [None] * n_chunks
        for ci in range(n_chunks):
            cur = ci % 2
            if ci >= 2:
                stores[ci - 2].wait()
            gathers[ci] = pltpu.async_copy(
                tok_hbm.at[idx_v.at[pl.ds(ci * chunk, chunk)]],
                rows[cur],
                gsem[cur],
            )
            if ci >= 1:
                gathers[ci - 1].wait()
                stores[ci - 1] = pltpu.async_copy(
                    rows[1 - cur],
                    out_hbm.at[pl.ds(base + (ci - 1) * chunk, chunk)],
                    ssem[1 - cur],
                )
        gathers[n_chunks - 1].wait()
        last = (n_chunks - 1) % 2
        stores[n_chunks - 1] = pltpu.async_copy(
            rows[last],
            out_hbm.at[pl.ds(base + (n_chunks - 1) * chunk, chunk)],
            ssem[last],
        )
        stores[n_chunks - 2].wait()
        stores[n_chunks - 1].wait()

    return k(x_flat, tok_embed)


def _addmm_bias(h_tok, pos_embed, W, b2d):
    m, k = h_tok.shape
    seq = pos_embed.shape[0]
    n = W.shape[1]
    tn = 256
    rep = m // seq

    rc = 512

    def mm_kernel(h_ref, p_ref, w_ref, b_ref, out_ref, h16_ref):
        @pl.when(pl.program_id(0) == 0)
        def _():
            for c in range(m // rc):
                pr = (c % (seq // rc)) * rc
                h16_ref[pl.ds(c * rc, rc), :] = (
                    h_ref[pl.ds(c * rc, rc), :] + p_ref[pl.ds(pr, rc), :]
                ).astype(jnp.bfloat16)

        w16 = w_ref[...].astype(jnp.bfloat16)
        mc = 1024
        for c in range(m // mc):
            out_ref[pl.ds(c * mc, mc), :] = (
                jnp.dot(
                    h16_ref[pl.ds(c * mc, mc), :],
                    w16,
                    preferred_element_type=jnp.float32,
                )
                + b_ref[...]
            )

    return pl.pallas_call(
        mm_kernel,
        grid=(n // tn,),
        in_specs=[
            pl.BlockSpec((m, k), lambda j: (0, 0)),
            pl.BlockSpec((seq, k), lambda j: (0, 0)),
            pl.BlockSpec((k, tn), lambda j: (0, j)),
            pl.BlockSpec((1, tn), lambda j: (0, j)),
        ],
        out_specs=pl.BlockSpec((m, tn), lambda j: (0, j)),
        out_shape=jax.ShapeDtypeStruct((m, n), jnp.float32),
        scratch_shapes=[pltpu.VMEM((m, k), jnp.bfloat16)],
        compiler_params=pltpu.CompilerParams(
            vmem_limit_bytes=96 * 1024 * 1024
        ),
    )(h_tok, pos_embed, W, b2d)


def kernel(x, tok_embed, pos_embed, W, b):
    batch, seq = x.shape
    vocab = W.shape[1]
    x_flat = x.reshape(-1).astype(jnp.int32)
    h_tok = _gather_rows(x_flat, tok_embed)
    logits = _addmm_bias(h_tok, pos_embed, W, b.reshape(1, vocab))
    return logits.reshape(batch, seq, vocab), None


# speedup vs baseline: 1.0616x; 1.0616x over previous
import functools

import jax
import jax.numpy as jnp
from jax import lax
from jax.experimental import pallas as pl
from jax.experimental.pallas import tpu as pltpu
from jax.experimental.pallas import tpu_sc as plsc


def _gather_rows(x_flat, tok_embed):
    n = x_flat.shape[0]
    embed = tok_embed.shape[1]
    info = plsc.get_sparse_core_info()
    nc, ns = info.num_cores, info.num_subcores
    nw = nc * ns
    rows_per_w = n // nw
    chunk = 32
    n_chunks = rows_per_w // chunk

    mesh = plsc.VectorSubcoreMesh(core_axis_name="c", subcore_axis_name="s")

    @functools.partial(
        pl.kernel,
        mesh=mesh,
        out_type=jax.ShapeDtypeStruct((n, embed), jnp.float32),
        scratch_types=[
            pltpu.VMEM((rows_per_w,), jnp.int32),
            pltpu.VMEM((chunk, embed), jnp.float32),
            pltpu.VMEM((chunk, embed), jnp.float32),
            pltpu.SemaphoreType.DMA,
            pltpu.SemaphoreType.DMA,
            pltpu.SemaphoreType.DMA,
            pltpu.SemaphoreType.DMA,
        ],
    )
    def k(x_hbm, tok_hbm, out_hbm, idx_v, rows0, rows1, g0, g1, s0, s1):
        wid = lax.axis_index("s") * nc + lax.axis_index("c")
        base = wid * rows_per_w
        pltpu.sync_copy(x_hbm.at[pl.ds(base, rows_per_w)], idx_v)
        rows = (rows0, rows1)
        gsem = (g0, g1)
        ssem = (s0, s1)
        gathers = [None] * n_chunks
        stores = [None] * n_chunks
        for ci in range(n_chunks):
            cur = ci % 2
            if ci >= 2:
                stores[ci - 2].wait()
            gathers[ci] = pltpu.async_copy(
                tok_hbm.at[idx_v.at[pl.ds(ci * chunk, chunk)]],
                rows[cur],
                gsem[cur],
            )
            if ci >= 1:
                gathers[ci - 1].wait()
                stores[ci - 1] = pltpu.async_copy(
                    rows[1 - cur],
                    out_hbm.at[pl.ds(base + (ci - 1) * chunk, chunk)],
                    ssem[1 - cur],
                )
        gathers[n_chunks - 1].wait()
        last = (n_chunks - 1) % 2
        stores[n_chunks - 1] = pltpu.async_copy(
            rows[last],
            out_hbm.at[pl.ds(base + (n_chunks - 1) * chunk, chunk)],
            ssem[last],
        )
        stores[n_chunks - 2].wait()
        stores[n_chunks - 1].wait()

    return k(x_flat, tok_embed)


def _add_cast(h_tok, pos_embed):
    m, k = h_tok.shape
    seq = pos_embed.shape[0]

    def ac_kernel(h_ref, p_ref, o_ref):
        rc = 512
        for c in range(m // rc):
            pr = (c % (seq // rc)) * rc
            o_ref[pl.ds(c * rc, rc), :] = (
                h_ref[pl.ds(c * rc, rc), :] + p_ref[pl.ds(pr, rc), :]
            ).astype(jnp.bfloat16)

    return pl.pallas_call(
        ac_kernel,
        in_specs=[
            pl.BlockSpec((m, k), lambda: (0, 0)),
            pl.BlockSpec((seq, k), lambda: (0, 0)),
        ],
        out_specs=pl.BlockSpec((m, k), lambda: (0, 0)),
        out_shape=jax.ShapeDtypeStruct((m, k), jnp.bfloat16),
    )(h_tok, pos_embed)


def _addmm_bias(h16, W, b2d):
    m, k = h16.shape
    n = W.shape[1]
    tn = 1280

    def mm_kernel(h_ref, w_ref, b_ref, out_ref):
        w16 = w_ref[...].astype(jnp.bfloat16)
        mc = 1024
        for c in range(m // mc):
            out_ref[pl.ds(c * mc, mc), :] = (
                jnp.dot(
                    h_ref[pl.ds(c * mc, mc), :],
                    w16,
                    preferred_element_type=jnp.float32,
                )
                + b_ref[...]
            )

    return pl.pallas_call(
        mm_kernel,
        grid=(n // tn,),
        in_specs=[
            pl.BlockSpec((m, k), lambda j: (0, 0)),
            pl.BlockSpec((k, tn), lambda j: (0, j)),
            pl.BlockSpec((1, tn), lambda j: (0, j)),
        ],
        out_specs=pl.BlockSpec((m, tn), lambda j: (0, j)),
        out_shape=jax.ShapeDtypeStruct((m, n), jnp.float32),
        compiler_params=pltpu.CompilerParams(
            vmem_limit_bytes=96 * 1024 * 1024
        ),
    )(h16, W, b2d)


def kernel(x, tok_embed, pos_embed, W, b):
    batch, seq = x.shape
    vocab = W.shape[1]
    x_flat = x.reshape(-1).astype(jnp.int32)
    h_tok = _gather_rows(x_flat, tok_embed)
    h16 = _add_cast(h_tok, pos_embed)
    logits = _addmm_bias(h16, W, b.reshape(1, vocab))
    return logits.reshape(batch, seq, vocab), None


# speedup vs baseline: 1.0633x; 1.0016x over previous
import functools

import jax
import jax.numpy as jnp
from jax import lax
from jax.experimental import pallas as pl
from jax.experimental.pallas import tpu as pltpu
from jax.experimental.pallas import tpu_sc as plsc


def _gather_rows(x_flat, tok_embed):
    n = x_flat.shape[0]
    embed = tok_embed.shape[1]
    info = plsc.get_sparse_core_info()
    nc, ns = info.num_cores, info.num_subcores
    nw = nc * ns
    rows_per_w = n // nw
    chunk = 32
    n_chunks = rows_per_w // chunk

    mesh = plsc.VectorSubcoreMesh(core_axis_name="c", subcore_axis_name="s")

    @functools.partial(
        pl.kernel,
        mesh=mesh,
        out_type=jax.ShapeDtypeStruct((n, embed), jnp.float32),
        scratch_types=[
            pltpu.VMEM((rows_per_w,), jnp.int32),
            pltpu.VMEM((chunk, embed), jnp.float32),
            pltpu.VMEM((chunk, embed), jnp.float32),
            pltpu.SemaphoreType.DMA,
            pltpu.SemaphoreType.DMA,
            pltpu.SemaphoreType.DMA,
            pltpu.SemaphoreType.DMA,
        ],
    )
    def k(x_hbm, tok_hbm, out_hbm, idx_v, rows0, rows1, g0, g1, s0, s1):
        wid = lax.axis_index("s") * nc + lax.axis_index("c")
        base = wid * rows_per_w
        pltpu.sync_copy(x_hbm.at[pl.ds(base, rows_per_w)], idx_v)
        rows = (rows0, rows1)
        gsem = (g0, g1)
        ssem = (s0, s1)
        gathers = [None] * n_chunks
        stores = [None] * n_chunks
        for ci in range(n_chunks):
            cur = ci % 2
            if ci >= 2:
                stores[ci - 2].wait()
            gathers[ci] = pltpu.async_copy(
                tok_hbm.at[idx_v.at[pl.ds(ci * chunk, chunk)]],
                rows[cur],
                gsem[cur],
            )
            if ci >= 1:
                gathers[ci - 1].wait()
                stores[ci - 1] = pltpu.async_copy(
                    rows[1 - cur],
                    out_hbm.at[pl.ds(base + (ci - 1) * chunk, chunk)],
                    ssem[1 - cur],
                )
        gathers[n_chunks - 1].wait()
        last = (n_chunks - 1) % 2
        stores[n_chunks - 1] = pltpu.async_copy(
            rows[last],
            out_hbm.at[pl.ds(base + (n_chunks - 1) * chunk, chunk)],
            ssem[last],
        )
        stores[n_chunks - 2].wait()
        stores[n_chunks - 1].wait()

    return k(x_flat, tok_embed)


def _add_cast(h_tok, pos_embed):
    m, k = h_tok.shape
    seq = pos_embed.shape[0]

    def ac_kernel(h_ref, p_ref, o_ref):
        rc = 512
        for c in range(m // rc):
            pr = (c % (seq // rc)) * rc
            o_ref[pl.ds(c * rc, rc), :] = (
                h_ref[pl.ds(c * rc, rc), :] + p_ref[pl.ds(pr, rc), :]
            ).astype(jnp.bfloat16)

    return pl.pallas_call(
        ac_kernel,
        in_specs=[
            pl.BlockSpec((m, k), lambda: (0, 0)),
            pl.BlockSpec((seq, k), lambda: (0, 0)),
        ],
        out_specs=pl.BlockSpec((m, k), lambda: (0, 0)),
        out_shape=jax.ShapeDtypeStruct((m, k), jnp.bfloat16),
    )(h_tok, pos_embed)


def _addmm_bias(h16, W, b2d):
    m, k = h16.shape
    n = W.shape[1]
    tn = 1280

    def mm_kernel(h_ref, w_ref, b_ref, out_ref):
        w16 = w_ref[...].astype(jnp.bfloat16)
        mc = 512
        for c in range(m // mc):
            out_ref[pl.ds(c * mc, mc), :] = (
                jnp.dot(
                    h_ref[pl.ds(c * mc, mc), :],
                    w16,
                    preferred_element_type=jnp.float32,
                )
                + b_ref[...]
            )

    return pl.pallas_call(
        mm_kernel,
        grid=(n // tn,),
        in_specs=[
            pl.BlockSpec((m, k), lambda j: (0, 0)),
            pl.BlockSpec((k, tn), lambda j: (0, j)),
            pl.BlockSpec((1, tn), lambda j: (0, j)),
        ],
        out_specs=pl.BlockSpec((m, tn), lambda j: (0, j)),
        out_shape=jax.ShapeDtypeStruct((m, n), jnp.float32),
        compiler_params=pltpu.CompilerParams(
            vmem_limit_bytes=96 * 1024 * 1024
        ),
    )(h16, W, b2d)


def kernel(x, tok_embed, pos_embed, W, b):
    batch, seq = x.shape
    vocab = W.shape[1]
    x_flat = x.reshape(-1).astype(jnp.int32)
    h_tok = _gather_rows(x_flat, tok_embed)
    h16 = _add_cast(h_tok, pos_embed)
    logits = _addmm_bias(h16, W, b.reshape(1, vocab))
    return logits.reshape(batch, seq, vocab), None
